# Initial kernel scaffold; baseline (speedup 1.0000x reference)
import functools

import jax
import jax.numpy as jnp
from jax import lax
from jax.experimental import pallas as pl
from jax.experimental.pallas import tpu as pltpu
from jax.experimental.pallas import tpu_sc as plsc

N_NODES = 10000
N_EDGES = 320000
NFEAT = 128
NHID = 128
NCLASS = 64
NLAYERS = 4
ALPHA = 0.1

NW = 32
EPW = N_EDGES // NW
CHUNK = 125
NCHUNK = EPW // CHUNK
ROWS_PER_TILE = N_NODES // 16
NSLC = NHID // 16

_ROW_BLK = 1000


def _in_layer_body(x_ref, w_ref, b_ref, o_ref):
    y = jnp.dot(x_ref[...], w_ref[...], preferred_element_type=jnp.float32)
    o_ref[...] = jnp.maximum(y + b_ref[...], 0.0)


def _combine_body(p_ref, h0_ref, o_ref):
    agg = p_ref[0] + p_ref[1]
    o_ref[...] = jnp.maximum((1.0 - ALPHA) * agg + ALPHA * h0_ref[...], 0.0)


def _out_layer_body(h_ref, w_ref, b_ref, o_ref):
    y = jnp.dot(h_ref[...], w_ref[...], preferred_element_type=jnp.float32)
    y = y + b_ref[...]
    m = jnp.max(y, axis=1, keepdims=True)
    s = jnp.sum(jnp.exp(y - m), axis=1, keepdims=True)
    o_ref[...] = (y - m) - jnp.log(s)


def _in_layer(x, W_in, b_in):
    grid = N_NODES // _ROW_BLK
    return pl.pallas_call(
        _in_layer_body,
        grid=(grid,),
        in_specs=[
            pl.BlockSpec((_ROW_BLK, NFEAT), lambda i: (i, 0)),
            pl.BlockSpec((NFEAT, NHID), lambda i: (0, 0)),
            pl.BlockSpec((1, NHID), lambda i: (0, 0)),
        ],
        out_specs=pl.BlockSpec((_ROW_BLK, NHID), lambda i: (i, 0)),
        out_shape=jax.ShapeDtypeStruct((N_NODES, NHID), jnp.float32),
    )(x, W_in, b_in.reshape(1, NHID))


def _combine(p, h0):
    grid = N_NODES // _ROW_BLK
    return pl.pallas_call(
        _combine_body,
        grid=(grid,),
        in_specs=[
            pl.BlockSpec((2, _ROW_BLK, NHID), lambda i: (0, i, 0)),
            pl.BlockSpec((_ROW_BLK, NHID), lambda i: (i, 0)),
        ],
        out_specs=pl.BlockSpec((_ROW_BLK, NHID), lambda i: (i, 0)),
        out_shape=jax.ShapeDtypeStruct((N_NODES, NHID), jnp.float32),
    )(p, h0)


def _out_layer(h, W_out, b_out):
    grid = N_NODES // _ROW_BLK
    return pl.pallas_call(
        _out_layer_body,
        grid=(grid,),
        in_specs=[
            pl.BlockSpec((_ROW_BLK, NHID), lambda i: (i, 0)),
            pl.BlockSpec((NHID, NCLASS), lambda i: (0, 0)),
            pl.BlockSpec((1, NCLASS), lambda i: (0, 0)),
        ],
        out_specs=pl.BlockSpec((_ROW_BLK, NCLASS), lambda i: (i, 0)),
        out_shape=jax.ShapeDtypeStruct((N_NODES, NCLASS), jnp.float32),
    )(h, W_out, b_out.reshape(1, NCLASS))


def _spmm_body(h_hbm, src_hbm, dst_hbm, w_hbm, out_hbm,
               src_v, dst_v, w_v, stage_v, acc_sh, sem):
    cid = lax.axis_index("c")
    sid = lax.axis_index("s")
    wid = sid * 2 + cid

    @pl.loop(0, CHUNK)
    def _zero_row(e):
        for s in range(NSLC):
            stage_v[e, pl.ds(s * 16, 16)] = jnp.zeros((16,), jnp.float32)

    for k in range(ROWS_PER_TILE // CHUNK):
        pltpu.sync_copy(
            stage_v, acc_sh.at[pl.ds(sid * ROWS_PER_TILE + k * CHUNK, CHUNK)])

    pltpu.sync_copy(src_hbm.at[wid], src_v)
    pltpu.sync_copy(dst_hbm.at[wid], dst_v)
    pltpu.sync_copy(w_hbm.at[wid], w_v)

    plsc.subcore_barrier()

    @pl.loop(0, NCHUNK)
    def _chunk(j):
        pltpu.async_copy(h_hbm.at[src_v.at[j]], stage_v, sem).wait()

        @pl.loop(0, CHUNK)
        def _scale(e):
            wv = jnp.full((16,), w_v[j, e], jnp.float32)
            for s in range(NSLC):
                sl = pl.ds(s * 16, 16)
                stage_v[e, sl] = stage_v[e, sl] * wv

        pltpu.sync_copy(stage_v, acc_sh.at[dst_v.at[j]], add=True)

    plsc.subcore_barrier()

    pltpu.sync_copy(
        acc_sh.at[pl.ds(sid * ROWS_PER_TILE, ROWS_PER_TILE)],
        out_hbm.at[cid, pl.ds(sid * ROWS_PER_TILE, ROWS_PER_TILE)])


_spmm_sc = functools.partial(
    pl.kernel,
    out_type=jax.ShapeDtypeStruct((2, N_NODES, NHID), jnp.float32),
    mesh=plsc.VectorSubcoreMesh(core_axis_name="c", subcore_axis_name="s"),
    scratch_types=[
        pltpu.VMEM((NCHUNK, CHUNK), jnp.int32),
        pltpu.VMEM((NCHUNK, CHUNK), jnp.int32),
        pltpu.VMEM((NCHUNK, CHUNK), jnp.float32),
        pltpu.VMEM((CHUNK, NHID), jnp.float32),
        pltpu.VMEM_SHARED((N_NODES, NHID), jnp.float32),
        pltpu.SemaphoreType.DMA,
    ],
)(_spmm_body)


def kernel(x, edge_index, edge_weight, W_in, b_in, W_out, b_out):
    src = edge_index[0].reshape(NW, NCHUNK, CHUNK)
    dst = edge_index[1].reshape(NW, NCHUNK, CHUNK)
    w = edge_weight.reshape(NW, NCHUNK, CHUNK)

    h0 = _in_layer(x, W_in, b_in)
    h = h0
    for _ in range(NLAYERS):
        p = _spmm_sc(h, src, dst, w)
        h = _combine(p, h0)
    return _out_layer(h, W_out, b_out)


# speedup vs baseline: 3.1426x; 3.1426x over previous
import functools

import jax
import jax.numpy as jnp
from jax import lax
from jax.experimental import pallas as pl
from jax.experimental.pallas import tpu as pltpu
from jax.experimental.pallas import tpu_sc as plsc

N_NODES = 10000
N_EDGES = 320000
NFEAT = 128
NHID = 128
NCLASS = 64
NLAYERS = 4
ALPHA = 0.1

NW = 32
CHUNK = 128
NCHUNK = 80
EPW = NCHUNK * CHUNK
E_PAD = NW * EPW
SUPER = 40
NSUPER = NCHUNK // SUPER
PAD_N = 10240
ROWS_PER_TILE = PAD_N // 16
NSLC = NHID // 16

_ROW_BLK = 1000


def _in_layer_body(x_ref, w_ref, b_ref, o_ref):
    y = jnp.dot(x_ref[...], w_ref[...], preferred_element_type=jnp.float32)
    o_ref[...] = jnp.maximum(y + b_ref[...], 0.0)


def _combine_body(p_ref, h0_ref, o_ref):
    agg = p_ref[0] + p_ref[1]
    o_ref[...] = jnp.maximum(
        (1.0 - ALPHA) * agg + ALPHA * h0_ref[...], 0.0)


def _out_layer_body(h_ref, w_ref, b_ref, o_ref):
    y = jnp.dot(h_ref[...], w_ref[...], preferred_element_type=jnp.float32)
    y = y + b_ref[...]
    m = jnp.max(y, axis=1, keepdims=True)
    s = jnp.sum(jnp.exp(y - m), axis=1, keepdims=True)
    o_ref[...] = (y - m) - jnp.log(s)


def _in_layer(x, W_in, b_in):
    grid = N_NODES // _ROW_BLK
    return pl.pallas_call(
        _in_layer_body,
        grid=(grid,),
        in_specs=[
            pl.BlockSpec((_ROW_BLK, NFEAT), lambda i: (i, 0)),
            pl.BlockSpec((NFEAT, NHID), lambda i: (0, 0)),
            pl.BlockSpec((1, NHID), lambda i: (0, 0)),
        ],
        out_specs=pl.BlockSpec((_ROW_BLK, NHID), lambda i: (i, 0)),
        out_shape=jax.ShapeDtypeStruct((N_NODES, NHID), jnp.float32),
    )(x, W_in, b_in.reshape(1, NHID))


def _combine(p, h0):
    grid = N_NODES // _ROW_BLK
    return pl.pallas_call(
        _combine_body,
        grid=(grid,),
        in_specs=[
            pl.BlockSpec((2, _ROW_BLK, NHID), lambda i: (0, i, 0)),
            pl.BlockSpec((_ROW_BLK, NHID), lambda i: (i, 0)),
        ],
        out_specs=pl.BlockSpec((_ROW_BLK, NHID), lambda i: (i, 0)),
        out_shape=jax.ShapeDtypeStruct((N_NODES, NHID), jnp.float32),
    )(p, h0)


def _out_layer(h, W_out, b_out):
    grid = N_NODES // _ROW_BLK
    return pl.pallas_call(
        _out_layer_body,
        grid=(grid,),
        in_specs=[
            pl.BlockSpec((_ROW_BLK, NHID), lambda i: (i, 0)),
            pl.BlockSpec((NHID, NCLASS), lambda i: (0, 0)),
            pl.BlockSpec((1, NCLASS), lambda i: (0, 0)),
        ],
        out_specs=pl.BlockSpec((_ROW_BLK, NCLASS), lambda i: (i, 0)),
        out_shape=jax.ShapeDtypeStruct((N_NODES, NCLASS), jnp.float32),
    )(h, W_out, b_out.reshape(1, NCLASS))


def _spmm_body(h_hbm, src_hbm, dst_hbm, w_hbm, out_hbm,
               src_v, dst_v, w_v, stage_v, acc_sh, sem):
    cid = lax.axis_index("c")
    sid = lax.axis_index("s")
    wid = sid * 2 + cid

    @pl.loop(0, CHUNK)
    def _zero_row(e):
        for s in range(NSLC):
            stage_v[e, pl.ds(s * 16, 16)] = jnp.zeros((16,), jnp.float32)

    for k in range(ROWS_PER_TILE // CHUNK):
        pltpu.sync_copy(
            stage_v, acc_sh.at[pl.ds(sid * ROWS_PER_TILE + k * CHUNK, CHUNK)])

    plsc.subcore_barrier()

    @pl.loop(0, NSUPER)
    def _super(b):
        sb = pl.ds(b * SUPER, SUPER)
        pltpu.sync_copy(src_hbm.at[wid, sb], src_v)
        pltpu.sync_copy(dst_hbm.at[wid, sb], dst_v)
        pltpu.sync_copy(w_hbm.at[wid, sb], w_v)

        @pl.loop(0, SUPER)
        def _chunk(j):
            pltpu.async_copy(h_hbm.at[src_v.at[j]], stage_v, sem).wait()

            @pl.loop(0, CHUNK // 16)
            def _scale(g):
                w16 = w_v[j, pl.ds(g * 16, 16)]
                for t in range(16):
                    wv = jnp.full((16,), w16[t], jnp.float32)
                    e = g * 16 + t
                    for s in range(NSLC):
                        sl = pl.ds(s * 16, 16)
                        stage_v[e, sl] = stage_v[e, sl] * wv

            pltpu.sync_copy(stage_v, acc_sh.at[dst_v.at[j]], add=True)

    plsc.subcore_barrier()

    pltpu.sync_copy(
        acc_sh.at[pl.ds(sid * ROWS_PER_TILE, ROWS_PER_TILE)],
        out_hbm.at[cid, pl.ds(sid * ROWS_PER_TILE, ROWS_PER_TILE)])


_spmm_sc = functools.partial(
    pl.kernel,
    out_type=jax.ShapeDtypeStruct((2, PAD_N, NHID), jnp.float32),
    mesh=plsc.VectorSubcoreMesh(core_axis_name="c", subcore_axis_name="s"),
    scratch_types=[
        pltpu.VMEM((SUPER, CHUNK), jnp.int32),
        pltpu.VMEM((SUPER, CHUNK), jnp.int32),
        pltpu.VMEM((SUPER, CHUNK), jnp.float32),
        pltpu.VMEM((CHUNK, NHID), jnp.float32),
        pltpu.VMEM_SHARED((PAD_N, NHID), jnp.float32),
        pltpu.SemaphoreType.DMA,
    ],
)(_spmm_body)


def kernel(x, edge_index, edge_weight, W_in, b_in, W_out, b_out):
    npad = E_PAD - N_EDGES
    src = jnp.concatenate(
        [edge_index[0], jnp.zeros((npad,), jnp.int32)]).reshape(
            NW, NCHUNK, CHUNK)
    dst = jnp.concatenate(
        [edge_index[1], jnp.full((npad,), N_NODES, jnp.int32)]).reshape(
            NW, NCHUNK, CHUNK)
    w = jnp.concatenate(
        [edge_weight, jnp.zeros((npad,), jnp.float32)]).reshape(
            NW, NCHUNK, CHUNK)

    h0 = _in_layer(x, W_in, b_in)
    h = h0
    for _ in range(NLAYERS):
        p = _spmm_sc(h, src, dst, w)
        h = _combine(p, h0)
    return _out_layer(h, W_out, b_out)


# speedup vs baseline: 3.4754x; 1.1059x over previous
import functools

import jax
import jax.numpy as jnp
from jax import lax
from jax.experimental import pallas as pl
from jax.experimental.pallas import tpu as pltpu
from jax.experimental.pallas import tpu_sc as plsc

N_NODES = 10000
N_EDGES = 320000
NFEAT = 128
NHID = 128
NCLASS = 64
NLAYERS = 4
ALPHA = 0.1

NW = 32
CHUNK = 128
NCHUNK = 80
EPW = NCHUNK * CHUNK
E_PAD = NW * EPW
SUPER = 16
NSUPER = NCHUNK // SUPER
PAD_N = 10240
ROWS_PER_TILE = PAD_N // 16
NSLC = NHID // 16

_ROW_BLK = 1000


def _in_layer_body(x_ref, w_ref, b_ref, o_ref):
    y = jnp.dot(x_ref[...], w_ref[...], preferred_element_type=jnp.float32)
    o_ref[...] = jnp.maximum(y + b_ref[...], 0.0)


def _combine_body(p_ref, h0_ref, o_ref):
    agg = p_ref[0] + p_ref[1]
    o_ref[...] = jnp.maximum(
        (1.0 - ALPHA) * agg + ALPHA * h0_ref[...], 0.0)


def _out_layer_body(h_ref, w_ref, b_ref, o_ref):
    y = jnp.dot(h_ref[...], w_ref[...], preferred_element_type=jnp.float32)
    y = y + b_ref[...]
    m = jnp.max(y, axis=1, keepdims=True)
    s = jnp.sum(jnp.exp(y - m), axis=1, keepdims=True)
    o_ref[...] = (y - m) - jnp.log(s)


def _in_layer(x, W_in, b_in):
    grid = N_NODES // _ROW_BLK
    return pl.pallas_call(
        _in_layer_body,
        grid=(grid,),
        in_specs=[
            pl.BlockSpec((_ROW_BLK, NFEAT), lambda i: (i, 0)),
            pl.BlockSpec((NFEAT, NHID), lambda i: (0, 0)),
            pl.BlockSpec((1, NHID), lambda i: (0, 0)),
        ],
        out_specs=pl.BlockSpec((_ROW_BLK, NHID), lambda i: (i, 0)),
        out_shape=jax.ShapeDtypeStruct((N_NODES, NHID), jnp.float32),
    )(x, W_in, b_in.reshape(1, NHID))


def _combine(p, h0):
    grid = N_NODES // _ROW_BLK
    return pl.pallas_call(
        _combine_body,
        grid=(grid,),
        in_specs=[
            pl.BlockSpec((2, _ROW_BLK, NHID), lambda i: (0, i, 0)),
            pl.BlockSpec((_ROW_BLK, NHID), lambda i: (i, 0)),
        ],
        out_specs=pl.BlockSpec((_ROW_BLK, NHID), lambda i: (i, 0)),
        out_shape=jax.ShapeDtypeStruct((N_NODES, NHID), jnp.float32),
    )(p, h0)


def _out_layer(h, W_out, b_out):
    grid = N_NODES // _ROW_BLK
    return pl.pallas_call(
        _out_layer_body,
        grid=(grid,),
        in_specs=[
            pl.BlockSpec((_ROW_BLK, NHID), lambda i: (i, 0)),
            pl.BlockSpec((NHID, NCLASS), lambda i: (0, 0)),
            pl.BlockSpec((1, NCLASS), lambda i: (0, 0)),
        ],
        out_specs=pl.BlockSpec((_ROW_BLK, NCLASS), lambda i: (i, 0)),
        out_shape=jax.ShapeDtypeStruct((N_NODES, NCLASS), jnp.float32),
    )(h, W_out, b_out.reshape(1, NCLASS))


def _spmm_body(h_hbm, src_hbm, dst_hbm, w_hbm, out_hbm,
               src_v, dst_v, w_v, stg0, stg1, acc_sh, sem):
    cid = lax.axis_index("c")
    sid = lax.axis_index("s")
    wid = sid * 2 + cid

    @pl.loop(0, CHUNK)
    def _zero_row(e):
        for s in range(NSLC):
            stg0[e, pl.ds(s * 16, 16)] = jnp.zeros((16,), jnp.float32)

    for k in range(ROWS_PER_TILE // CHUNK):
        pltpu.sync_copy(
            stg0, acc_sh.at[pl.ds(sid * ROWS_PER_TILE + k * CHUNK, CHUNK)])

    plsc.subcore_barrier()

    def _scale_chunk(buf, j):
        @pl.loop(0, CHUNK // 16)
        def _scale(g):
            w16 = w_v[j, pl.ds(g * 16, 16)]
            for t in range(16):
                wv = jnp.full((16,), w16[t], jnp.float32)
                e = g * 16 + t
                for s in range(NSLC):
                    sl = pl.ds(s * 16, 16)
                    buf[e, sl] = buf[e, sl] * wv

    @pl.loop(0, NSUPER)
    def _super(b):
        sb = pl.ds(pl.multiple_of(b * SUPER, 8), SUPER)
        pltpu.sync_copy(src_hbm.at[wid, sb], src_v)
        pltpu.sync_copy(dst_hbm.at[wid, sb], dst_v)
        pltpu.sync_copy(w_hbm.at[wid, sb], w_v)

        pltpu.async_copy(h_hbm.at[src_v.at[0]], stg0, sem)

        @pl.loop(0, SUPER // 2)
        def _pair(jj):
            j0 = jj * 2
            pltpu.make_async_copy(h_hbm.at[src_v.at[j0]], stg0, sem).wait()
            pltpu.async_copy(h_hbm.at[src_v.at[j0 + 1]], stg1, sem)
            _scale_chunk(stg0, j0)
            pltpu.sync_copy(stg0, acc_sh.at[dst_v.at[j0]], add=True)

            nxt = jnp.minimum(j0 + 2, SUPER - 1)
            pltpu.make_async_copy(h_hbm.at[src_v.at[j0 + 1]], stg1, sem).wait()
            pltpu.async_copy(h_hbm.at[src_v.at[nxt]], stg0, sem)
            _scale_chunk(stg1, j0 + 1)
            pltpu.sync_copy(stg1, acc_sh.at[dst_v.at[j0 + 1]], add=True)

        pltpu.make_async_copy(h_hbm.at[src_v.at[0]], stg0, sem).wait()

    plsc.subcore_barrier()

    pltpu.sync_copy(
        acc_sh.at[pl.ds(sid * ROWS_PER_TILE, ROWS_PER_TILE)],
        out_hbm.at[cid, pl.ds(sid * ROWS_PER_TILE, ROWS_PER_TILE)])


_spmm_sc = functools.partial(
    pl.kernel,
    out_type=jax.ShapeDtypeStruct((2, PAD_N, NHID), jnp.float32),
    mesh=plsc.VectorSubcoreMesh(core_axis_name="c", subcore_axis_name="s"),
    scratch_types=[
        pltpu.VMEM((SUPER, CHUNK), jnp.int32),
        pltpu.VMEM((SUPER, CHUNK), jnp.int32),
        pltpu.VMEM((SUPER, CHUNK), jnp.float32),
        pltpu.VMEM((CHUNK, NHID), jnp.float32),
        pltpu.VMEM((CHUNK, NHID), jnp.float32),
        pltpu.VMEM_SHARED((PAD_N, NHID), jnp.float32),
        pltpu.SemaphoreType.DMA,
    ],
)(_spmm_body)


def kernel(x, edge_index, edge_weight, W_in, b_in, W_out, b_out):
    npad = E_PAD - N_EDGES
    src = jnp.concatenate(
        [edge_index[0], jnp.zeros((npad,), jnp.int32)]).reshape(
            NW, NCHUNK, CHUNK)
    dst = jnp.concatenate(
        [edge_index[1], jnp.full((npad,), N_NODES, jnp.int32)]).reshape(
            NW, NCHUNK, CHUNK)
    w = jnp.concatenate(
        [edge_weight, jnp.zeros((npad,), jnp.float32)]).reshape(
            NW, NCHUNK, CHUNK)

    h0 = _in_layer(x, W_in, b_in)
    h = h0
    for _ in range(NLAYERS):
        p = _spmm_sc(h, src, dst, w)
        h = _combine(p, h0)
    return _out_layer(h, W_out, b_out)


# speedup vs baseline: 10.2873x; 2.9601x over previous
import functools

import jax
import jax.numpy as jnp
from jax import lax
from jax.experimental import pallas as pl
from jax.experimental.pallas import tpu as pltpu
from jax.experimental.pallas import tpu_sc as plsc

N_NODES = 10000
N_EDGES = 320000
NFEAT = 128
NHID = 128
NCLASS = 64
NLAYERS = 4
ALPHA = 0.1

NW = 32
CHUNK = 128
NCHUNK = 80
EPW = NCHUNK * CHUNK
E_PAD = NW * EPW
SUPER = 16
NSUPER = NCHUNK // SUPER
PAD_N = 10240
ROWS_PER_TILE = PAD_N // 16
NSLC = NHID // 16

_ROW_BLK = 1000


def _in_layer_body(x_ref, w_ref, b_ref, o_ref):
    y = jnp.dot(x_ref[...], w_ref[...], preferred_element_type=jnp.float32)
    o_ref[...] = jnp.maximum(y + b_ref[...], 0.0)


def _combine_body(p_ref, h0_ref, o_ref):
    agg = p_ref[0] + p_ref[1]
    o_ref[...] = jnp.maximum(
        (1.0 - ALPHA) * agg + ALPHA * h0_ref[...], 0.0)


def _out_layer_body(h_ref, w_ref, b_ref, o_ref):
    y = jnp.dot(h_ref[...], w_ref[...], preferred_element_type=jnp.float32)
    y = y + b_ref[...]
    m = jnp.max(y, axis=1, keepdims=True)
    s = jnp.sum(jnp.exp(y - m), axis=1, keepdims=True)
    o_ref[...] = (y - m) - jnp.log(s)


def _in_layer(x, W_in, b_in):
    grid = N_NODES // _ROW_BLK
    return pl.pallas_call(
        _in_layer_body,
        grid=(grid,),
        in_specs=[
            pl.BlockSpec((_ROW_BLK, NFEAT), lambda i: (i, 0)),
            pl.BlockSpec((NFEAT, NHID), lambda i: (0, 0)),
            pl.BlockSpec((1, NHID), lambda i: (0, 0)),
        ],
        out_specs=pl.BlockSpec((_ROW_BLK, NHID), lambda i: (i, 0)),
        out_shape=jax.ShapeDtypeStruct((N_NODES, NHID), jnp.float32),
    )(x, W_in, b_in.reshape(1, NHID))


def _combine(p, h0):
    grid = N_NODES // _ROW_BLK
    return pl.pallas_call(
        _combine_body,
        grid=(grid,),
        in_specs=[
            pl.BlockSpec((2, _ROW_BLK, NHID), lambda i: (0, i, 0)),
            pl.BlockSpec((_ROW_BLK, NHID), lambda i: (i, 0)),
        ],
        out_specs=pl.BlockSpec((_ROW_BLK, NHID), lambda i: (i, 0)),
        out_shape=jax.ShapeDtypeStruct((N_NODES, NHID), jnp.float32),
    )(p, h0)


def _out_layer(h, W_out, b_out):
    grid = N_NODES // _ROW_BLK
    return pl.pallas_call(
        _out_layer_body,
        grid=(grid,),
        in_specs=[
            pl.BlockSpec((_ROW_BLK, NHID), lambda i: (i, 0)),
            pl.BlockSpec((NHID, NCLASS), lambda i: (0, 0)),
            pl.BlockSpec((1, NCLASS), lambda i: (0, 0)),
        ],
        out_specs=pl.BlockSpec((_ROW_BLK, NCLASS), lambda i: (i, 0)),
        out_shape=jax.ShapeDtypeStruct((N_NODES, NCLASS), jnp.float32),
    )(h, W_out, b_out.reshape(1, NCLASS))


def _spmm_body(h_hbm, src_hbm, dst_hbm, w_hbm, out_hbm,
               src_v, dst_v, w_v, stg0, stg1, acc_sh, sem):
    cid = lax.axis_index("c")
    sid = lax.axis_index("s")
    wid = sid * 2 + cid

    @pl.loop(0, CHUNK)
    def _zero_row(e):
        for s in range(NSLC):
            stg0[e, pl.ds(s * 16, 16)] = jnp.zeros((16,), jnp.float32)

    for k in range(ROWS_PER_TILE // CHUNK):
        pltpu.sync_copy(
            stg0, acc_sh.at[pl.ds(sid * ROWS_PER_TILE + k * CHUNK, CHUNK)])

    plsc.subcore_barrier()

    def _scale_chunk(buf, j):
        @pl.loop(0, CHUNK // 16)
        def _scale(g):
            w16 = w_v[j, pl.ds(g * 16, 16)]
            for t in range(16):
                wv = jnp.full((16,), w16[t], jnp.float32)
                e = g * 16 + t
                for s in range(NSLC):
                    sl = pl.ds(s * 16, 16)
                    buf[e, sl] = buf[e, sl] * wv

    @pl.loop(0, NSUPER)
    def _super(b):
        sb = pl.ds(pl.multiple_of(b * SUPER, 8), SUPER)
        pltpu.sync_copy(src_hbm.at[wid, sb], src_v)
        pltpu.sync_copy(dst_hbm.at[wid, sb], dst_v)
        pltpu.sync_copy(w_hbm.at[wid, sb], w_v)

        pltpu.async_copy(h_hbm.at[src_v.at[0]], stg0, sem)

        @pl.loop(0, SUPER // 2)
        def _pair(jj):
            j0 = jj * 2
            pltpu.make_async_copy(h_hbm.at[src_v.at[j0]], stg0, sem).wait()
            pltpu.async_copy(h_hbm.at[src_v.at[j0 + 1]], stg1, sem)
            _scale_chunk(stg0, j0)
            pltpu.sync_copy(stg0, acc_sh.at[dst_v.at[j0]], add=True)

            nxt = jnp.minimum(j0 + 2, SUPER - 1)
            pltpu.make_async_copy(h_hbm.at[src_v.at[j0 + 1]], stg1, sem).wait()
            pltpu.async_copy(h_hbm.at[src_v.at[nxt]], stg0, sem)
            _scale_chunk(stg1, j0 + 1)
            pltpu.sync_copy(stg1, acc_sh.at[dst_v.at[j0 + 1]], add=True)

        pltpu.make_async_copy(h_hbm.at[src_v.at[0]], stg0, sem).wait()

    plsc.subcore_barrier()

    pltpu.sync_copy(
        acc_sh.at[pl.ds(sid * ROWS_PER_TILE, ROWS_PER_TILE)],
        out_hbm.at[cid, pl.ds(sid * ROWS_PER_TILE, ROWS_PER_TILE)])


_spmm_sc = functools.partial(
    pl.kernel,
    out_type=jax.ShapeDtypeStruct((2, PAD_N, NHID), jnp.float32),
    mesh=plsc.VectorSubcoreMesh(core_axis_name="c", subcore_axis_name="s"),
    scratch_types=[
        pltpu.VMEM((SUPER, CHUNK), jnp.int32),
        pltpu.VMEM((SUPER, CHUNK), jnp.int32),
        pltpu.VMEM((SUPER, CHUNK), jnp.float32),
        pltpu.VMEM((CHUNK, NHID), jnp.float32),
        pltpu.VMEM((CHUNK, NHID), jnp.float32),
        pltpu.VMEM_SHARED((PAD_N, NHID), jnp.float32),
        pltpu.SemaphoreType.DMA,
    ],
)(_spmm_body)


def kernel(x, edge_index, edge_weight, W_in, b_in, W_out, b_out):
    npad = E_PAD - N_EDGES
    iota = jnp.arange(npad, dtype=jnp.int32)
    src = jnp.concatenate(
        [edge_index[0], iota % N_NODES]).reshape(
            NW, NCHUNK, CHUNK)
    dst = jnp.concatenate(
        [edge_index[1], N_NODES + iota % (PAD_N - N_NODES)]).reshape(
            NW, NCHUNK, CHUNK)
    w = jnp.concatenate(
        [edge_weight, jnp.zeros((npad,), jnp.float32)]).reshape(
            NW, NCHUNK, CHUNK)

    h0 = _in_layer(x, W_in, b_in)
    h = h0
    for _ in range(NLAYERS):
        p = _spmm_sc(h, src, dst, w)
        h = _combine(p, h0)
    return _out_layer(h, W_out, b_out)


# speedup vs baseline: 10.4275x; 1.0136x over previous
import functools

import jax
import jax.numpy as jnp
from jax import lax
from jax.experimental import pallas as pl
from jax.experimental.pallas import tpu as pltpu
from jax.experimental.pallas import tpu_sc as plsc

N_NODES = 10000
N_EDGES = 320000
NFEAT = 128
NHID = 128
NCLASS = 64
NLAYERS = 4
ALPHA = 0.1

NW = 32
CHUNK = 128
NCHUNK = 80
EPW = NCHUNK * CHUNK
E_PAD = NW * EPW
SUPER = 16
NSUPER = NCHUNK // SUPER
PAD_N = 10240
ROWS_PER_TILE = PAD_N // 16
NSLC = NHID // 16

_ROW_BLK = 1000


def _in_layer_body(x_ref, w_ref, b_ref, o_ref):
    y = jnp.dot(x_ref[...], w_ref[...], preferred_element_type=jnp.float32)
    o_ref[...] = jnp.maximum(y + b_ref[...], 0.0)


def _combine_body(p_ref, h0_ref, o_ref):
    agg = p_ref[0] + p_ref[1]
    o_ref[...] = jnp.maximum(
        (1.0 - ALPHA) * agg + ALPHA * h0_ref[...], 0.0)


def _out_layer_body(h_ref, w_ref, b_ref, o_ref):
    y = jnp.dot(h_ref[...], w_ref[...], preferred_element_type=jnp.float32)
    y = y + b_ref[...]
    m = jnp.max(y, axis=1, keepdims=True)
    s = jnp.sum(jnp.exp(y - m), axis=1, keepdims=True)
    o_ref[...] = (y - m) - jnp.log(s)


def _in_layer(x, W_in, b_in):
    grid = N_NODES // _ROW_BLK
    return pl.pallas_call(
        _in_layer_body,
        grid=(grid,),
        in_specs=[
            pl.BlockSpec((_ROW_BLK, NFEAT), lambda i: (i, 0)),
            pl.BlockSpec((NFEAT, NHID), lambda i: (0, 0)),
            pl.BlockSpec((1, NHID), lambda i: (0, 0)),
        ],
        out_specs=pl.BlockSpec((_ROW_BLK, NHID), lambda i: (i, 0)),
        out_shape=jax.ShapeDtypeStruct((N_NODES, NHID), jnp.float32),
    )(x, W_in, b_in.reshape(1, NHID))


def _combine(p, h0):
    grid = N_NODES // _ROW_BLK
    return pl.pallas_call(
        _combine_body,
        grid=(grid,),
        in_specs=[
            pl.BlockSpec((2, _ROW_BLK, NHID), lambda i: (0, i, 0)),
            pl.BlockSpec((_ROW_BLK, NHID), lambda i: (i, 0)),
        ],
        out_specs=pl.BlockSpec((_ROW_BLK, NHID), lambda i: (i, 0)),
        out_shape=jax.ShapeDtypeStruct((N_NODES, NHID), jnp.float32),
    )(p, h0)


def _out_layer(h, W_out, b_out):
    grid = N_NODES // _ROW_BLK
    return pl.pallas_call(
        _out_layer_body,
        grid=(grid,),
        in_specs=[
            pl.BlockSpec((_ROW_BLK, NHID), lambda i: (i, 0)),
            pl.BlockSpec((NHID, NCLASS), lambda i: (0, 0)),
            pl.BlockSpec((1, NCLASS), lambda i: (0, 0)),
        ],
        out_specs=pl.BlockSpec((_ROW_BLK, NCLASS), lambda i: (i, 0)),
        out_shape=jax.ShapeDtypeStruct((N_NODES, NCLASS), jnp.float32),
    )(h, W_out, b_out.reshape(1, NCLASS))


def _spmm_body(h_hbm, src_hbm, dst_hbm, w_hbm, out_hbm,
               src_v, dst_v, w_v, stg0, stg1, acc_sh,
               gsem0, gsem1, csem0, csem1):
    cid = lax.axis_index("c")
    sid = lax.axis_index("s")
    wid = sid * 2 + cid

    @pl.loop(0, CHUNK)
    def _zero_row(e):
        for s in range(NSLC):
            stg0[e, pl.ds(s * 16, 16)] = jnp.zeros((16,), jnp.float32)

    for k in range(ROWS_PER_TILE // CHUNK):
        pltpu.sync_copy(
            stg0, acc_sh.at[pl.ds(sid * ROWS_PER_TILE + k * CHUNK, CHUNK)])

    plsc.subcore_barrier()

    def _scale_chunk(buf, j):
        @pl.loop(0, CHUNK // 16)
        def _scale(g):
            w16 = w_v[j, pl.ds(g * 16, 16)]
            for t in range(16):
                wv = jnp.full((16,), w16[t], jnp.float32)
                e = g * 16 + t
                for s in range(NSLC):
                    sl = pl.ds(s * 16, 16)
                    buf[e, sl] = buf[e, sl] * wv

    def _gather(j, buf, gsem):
        return pltpu.make_async_copy(h_hbm.at[src_v.at[j]], buf, gsem)

    def _scatter(j, buf, csem):
        return pltpu.make_async_copy(buf, acc_sh.at[dst_v.at[j]], csem)

    @pl.loop(0, NSUPER)
    def _super(b):
        sb = pl.ds(pl.multiple_of(b * SUPER, 8), SUPER)
        pltpu.sync_copy(src_hbm.at[wid, sb], src_v)
        pltpu.sync_copy(dst_hbm.at[wid, sb], dst_v)
        pltpu.sync_copy(w_hbm.at[wid, sb], w_v)

        _gather(0, stg0, gsem0).start()
        _gather(0, stg0, gsem0).wait()
        _gather(1, stg1, gsem1).start()
        _scale_chunk(stg0, 0)
        _scatter(0, stg0, csem0).start(add=True)

        @pl.loop(0, (SUPER - 2) // 2)
        def _pair(jj):
            j1 = jj * 2 + 1
            _scatter(j1 - 1, stg0, csem0).wait()
            _gather(j1 + 1, stg0, gsem0).start()
            _gather(j1, stg1, gsem1).wait()
            _scale_chunk(stg1, j1)
            _scatter(j1, stg1, csem1).start(add=True)

            j2 = j1 + 1
            _scatter(j2 - 1, stg1, csem1).wait()
            _gather(j2 + 1, stg1, gsem1).start()
            _gather(j2, stg0, gsem0).wait()
            _scale_chunk(stg0, j2)
            _scatter(j2, stg0, csem0).start(add=True)

        _scatter(SUPER - 2, stg0, csem0).wait()
        _gather(SUPER - 1, stg1, gsem1).wait()
        _scale_chunk(stg1, SUPER - 1)
        _scatter(SUPER - 1, stg1, csem1).start(add=True)
        _scatter(SUPER - 1, stg1, csem1).wait()

    plsc.subcore_barrier()

    pltpu.sync_copy(
        acc_sh.at[pl.ds(sid * ROWS_PER_TILE, ROWS_PER_TILE)],
        out_hbm.at[cid, pl.ds(sid * ROWS_PER_TILE, ROWS_PER_TILE)])


_spmm_sc = functools.partial(
    pl.kernel,
    out_type=jax.ShapeDtypeStruct((2, PAD_N, NHID), jnp.float32),
    mesh=plsc.VectorSubcoreMesh(core_axis_name="c", subcore_axis_name="s"),
    scratch_types=[
        pltpu.VMEM((SUPER, CHUNK), jnp.int32),
        pltpu.VMEM((SUPER, CHUNK), jnp.int32),
        pltpu.VMEM((SUPER, CHUNK), jnp.float32),
        pltpu.VMEM((CHUNK, NHID), jnp.float32),
        pltpu.VMEM((CHUNK, NHID), jnp.float32),
        pltpu.VMEM_SHARED((PAD_N, NHID), jnp.float32),
        pltpu.SemaphoreType.DMA,
        pltpu.SemaphoreType.DMA,
        pltpu.SemaphoreType.DMA,
        pltpu.SemaphoreType.DMA,
    ],
)(_spmm_body)


def kernel(x, edge_index, edge_weight, W_in, b_in, W_out, b_out):
    npad = E_PAD - N_EDGES
    iota = jnp.arange(npad, dtype=jnp.int32)
    src = jnp.concatenate(
        [edge_index[0], iota % N_NODES]).reshape(
            NW, NCHUNK, CHUNK)
    dst = jnp.concatenate(
        [edge_index[1], N_NODES + iota % (PAD_N - N_NODES)]).reshape(
            NW, NCHUNK, CHUNK)
    w = jnp.concatenate(
        [edge_weight, jnp.zeros((npad,), jnp.float32)]).reshape(
            NW, NCHUNK, CHUNK)

    h0 = _in_layer(x, W_in, b_in)
    h = h0
    for _ in range(NLAYERS):
        p = _spmm_sc(h, src, dst, w)
        h = _combine(p, h0)
    return _out_layer(h, W_out, b_out)


# speedup vs baseline: 10.4721x; 1.0043x over previous
import functools

import jax
import jax.numpy as jnp
from jax import lax
from jax.experimental import pallas as pl
from jax.experimental.pallas import tpu as pltpu
from jax.experimental.pallas import tpu_sc as plsc

N_NODES = 10000
N_EDGES = 320000
NFEAT = 128
NHID = 128
NCLASS = 64
NLAYERS = 4
ALPHA = 0.1

NW = 32
CHUNK = 128
NCHUNK = 80
EPW = NCHUNK * CHUNK
E_PAD = NW * EPW
SUPER = 16
NSUPER = NCHUNK // SUPER
PAD_N = 10240
ROWS_PER_TILE = PAD_N // 16
NSLC = NHID // 16

_ROW_BLK = 1000


def _in_layer_body(x_ref, w_ref, b_ref, o_ref):
    y = jnp.dot(x_ref[...], w_ref[...], preferred_element_type=jnp.float32)
    o_ref[...] = jnp.maximum(y + b_ref[...], 0.0)


def _combine_body(p_ref, h0_ref, o_ref):
    agg = p_ref[0] + p_ref[1]
    o_ref[...] = jnp.maximum(
        (1.0 - ALPHA) * agg + ALPHA * h0_ref[...], 0.0)


def _out_layer_body(h_ref, w_ref, b_ref, o_ref):
    y = jnp.dot(h_ref[...], w_ref[...], preferred_element_type=jnp.float32)
    y = y + b_ref[...]
    m = jnp.max(y, axis=1, keepdims=True)
    s = jnp.sum(jnp.exp(y - m), axis=1, keepdims=True)
    o_ref[...] = (y - m) - jnp.log(s)


def _in_layer(x, W_in, b_in):
    grid = N_NODES // _ROW_BLK
    return pl.pallas_call(
        _in_layer_body,
        grid=(grid,),
        in_specs=[
            pl.BlockSpec((_ROW_BLK, NFEAT), lambda i: (i, 0)),
            pl.BlockSpec((NFEAT, NHID), lambda i: (0, 0)),
            pl.BlockSpec((1, NHID), lambda i: (0, 0)),
        ],
        out_specs=pl.BlockSpec((_ROW_BLK, NHID), lambda i: (i, 0)),
        out_shape=jax.ShapeDtypeStruct((N_NODES, NHID), jnp.float32),
    )(x, W_in, b_in.reshape(1, NHID))


def _combine(p, h0):
    grid = N_NODES // _ROW_BLK
    return pl.pallas_call(
        _combine_body,
        grid=(grid,),
        in_specs=[
            pl.BlockSpec((2, _ROW_BLK, NHID), lambda i: (0, i, 0)),
            pl.BlockSpec((_ROW_BLK, NHID), lambda i: (i, 0)),
        ],
        out_specs=pl.BlockSpec((_ROW_BLK, NHID), lambda i: (i, 0)),
        out_shape=jax.ShapeDtypeStruct((N_NODES, NHID), jnp.float32),
    )(p, h0)


def _out_layer(h, W_out, b_out):
    grid = N_NODES // _ROW_BLK
    return pl.pallas_call(
        _out_layer_body,
        grid=(grid,),
        in_specs=[
            pl.BlockSpec((_ROW_BLK, NHID), lambda i: (i, 0)),
            pl.BlockSpec((NHID, NCLASS), lambda i: (0, 0)),
            pl.BlockSpec((1, NCLASS), lambda i: (0, 0)),
        ],
        out_specs=pl.BlockSpec((_ROW_BLK, NCLASS), lambda i: (i, 0)),
        out_shape=jax.ShapeDtypeStruct((N_NODES, NCLASS), jnp.float32),
    )(h, W_out, b_out.reshape(1, NCLASS))


def _spmm_body(h_hbm, src_hbm, dst_hbm, w_hbm, out_hbm,
               src_v, dst_v, w_v, stg0, stg1, acc_sh,
               gsem0a, gsem0b, gsem1a, gsem1b, csem0, csem1):
    gsem0 = (gsem0a, gsem0b)
    gsem1 = (gsem1a, gsem1b)
    cid = lax.axis_index("c")
    sid = lax.axis_index("s")
    wid = sid * 2 + cid

    @pl.loop(0, CHUNK)
    def _zero_row(e):
        for s in range(NSLC):
            stg0[e, pl.ds(s * 16, 16)] = jnp.zeros((16,), jnp.float32)

    for k in range(ROWS_PER_TILE // CHUNK):
        pltpu.sync_copy(
            stg0, acc_sh.at[pl.ds(sid * ROWS_PER_TILE + k * CHUNK, CHUNK)])

    plsc.subcore_barrier()

    def _scale_chunk(buf, j):
        @pl.loop(0, CHUNK // 16)
        def _scale(g):
            w16 = w_v[j, pl.ds(g * 16, 16)]
            for t in range(16):
                wv = jnp.full((16,), w16[t], jnp.float32)
                e = g * 16 + t
                for s in range(NSLC):
                    sl = pl.ds(s * 16, 16)
                    buf[e, sl] = buf[e, sl] * wv

    H2 = CHUNK // 2

    class _gather:
        def __init__(self, j, buf, gsem):
            self.a = pltpu.make_async_copy(
                h_hbm.at[src_v.at[j, pl.ds(0, H2)]], buf.at[pl.ds(0, H2)],
                gsem[0])
            self.b = pltpu.make_async_copy(
                h_hbm.at[src_v.at[j, pl.ds(H2, H2)]], buf.at[pl.ds(H2, H2)],
                gsem[1])

        def start(self):
            self.a.start()
            self.b.start()

        def wait(self):
            self.a.wait()
            self.b.wait()

    def _scatter(j, buf, csem):
        return pltpu.make_async_copy(buf, acc_sh.at[dst_v.at[j]], csem)

    @pl.loop(0, NSUPER)
    def _super(b):
        sb = pl.ds(pl.multiple_of(b * SUPER, 8), SUPER)
        pltpu.sync_copy(src_hbm.at[wid, sb], src_v)
        pltpu.sync_copy(dst_hbm.at[wid, sb], dst_v)
        pltpu.sync_copy(w_hbm.at[wid, sb], w_v)

        _gather(0, stg0, gsem0).start()
        _gather(0, stg0, gsem0).wait()
        _gather(1, stg1, gsem1).start()
        _scale_chunk(stg0, 0)
        _scatter(0, stg0, csem0).start(add=True)

        @pl.loop(0, (SUPER - 2) // 2)
        def _pair(jj):
            j1 = jj * 2 + 1
            _scatter(j1 - 1, stg0, csem0).wait()
            _gather(j1 + 1, stg0, gsem0).start()
            _gather(j1, stg1, gsem1).wait()
            _scale_chunk(stg1, j1)
            _scatter(j1, stg1, csem1).start(add=True)

            j2 = j1 + 1
            _scatter(j2 - 1, stg1, csem1).wait()
            _gather(j2 + 1, stg1, gsem1).start()
            _gather(j2, stg0, gsem0).wait()
            _scale_chunk(stg0, j2)
            _scatter(j2, stg0, csem0).start(add=True)

        _scatter(SUPER - 2, stg0, csem0).wait()
        _gather(SUPER - 1, stg1, gsem1).wait()
        _scale_chunk(stg1, SUPER - 1)
        _scatter(SUPER - 1, stg1, csem1).start(add=True)
        _scatter(SUPER - 1, stg1, csem1).wait()

    plsc.subcore_barrier()

    pltpu.sync_copy(
        acc_sh.at[pl.ds(sid * ROWS_PER_TILE, ROWS_PER_TILE)],
        out_hbm.at[cid, pl.ds(sid * ROWS_PER_TILE, ROWS_PER_TILE)])


_spmm_sc = functools.partial(
    pl.kernel,
    out_type=jax.ShapeDtypeStruct((2, PAD_N, NHID), jnp.float32),
    mesh=plsc.VectorSubcoreMesh(core_axis_name="c", subcore_axis_name="s"),
    scratch_types=[
        pltpu.VMEM((SUPER, CHUNK), jnp.int32),
        pltpu.VMEM((SUPER, CHUNK), jnp.int32),
        pltpu.VMEM((SUPER, CHUNK), jnp.float32),
        pltpu.VMEM((CHUNK, NHID), jnp.float32),
        pltpu.VMEM((CHUNK, NHID), jnp.float32),
        pltpu.VMEM_SHARED((PAD_N, NHID), jnp.float32),
        pltpu.SemaphoreType.DMA,
        pltpu.SemaphoreType.DMA,
        pltpu.SemaphoreType.DMA,
        pltpu.SemaphoreType.DMA,
        pltpu.SemaphoreType.DMA,
        pltpu.SemaphoreType.DMA,
    ],
)(_spmm_body)


def kernel(x, edge_index, edge_weight, W_in, b_in, W_out, b_out):
    npad = E_PAD - N_EDGES
    iota = jnp.arange(npad, dtype=jnp.int32)
    src = jnp.concatenate(
        [edge_index[0], iota % N_NODES]).reshape(
            NW, NCHUNK, CHUNK)
    dst = jnp.concatenate(
        [edge_index[1], N_NODES + iota % (PAD_N - N_NODES)]).reshape(
            NW, NCHUNK, CHUNK)
    w = jnp.concatenate(
        [edge_weight, jnp.zeros((npad,), jnp.float32)]).reshape(
            NW, NCHUNK, CHUNK)

    h0 = _in_layer(x, W_in, b_in)
    h = h0
    for _ in range(NLAYERS):
        p = _spmm_sc(h, src, dst, w)
        h = _combine(p, h0)
    return _out_layer(h, W_out, b_out)


# speedup vs baseline: 11.0621x; 1.0563x over previous
import functools

import jax
import jax.numpy as jnp
from jax import lax
from jax.experimental import pallas as pl
from jax.experimental.pallas import tpu as pltpu
from jax.experimental.pallas import tpu_sc as plsc

N_NODES = 10000
N_EDGES = 320000
NFEAT = 128
NHID = 128
NCLASS = 64
NLAYERS = 4
ALPHA = 0.1

NW = 32
CHUNK = 128
NCHUNK = 80
EPW = NCHUNK * CHUNK
E_PAD = NW * EPW
SUPER = 40
NSUPER = NCHUNK // SUPER
PAD_N = 10240
ROWS_PER_TILE = PAD_N // 16
NSLC = NHID // 16

_ROW_BLK = 1000


def _in_layer_body(x_ref, w_ref, b_ref, o_ref):
    y = jnp.dot(x_ref[...], w_ref[...], preferred_element_type=jnp.float32)
    o_ref[...] = jnp.maximum(y + b_ref[...], 0.0)


def _combine_body(p_ref, h0_ref, o_ref):
    agg = p_ref[0] + p_ref[1]
    o_ref[...] = jnp.maximum(
        (1.0 - ALPHA) * agg + ALPHA * h0_ref[...], 0.0)


def _out_layer_body(h_ref, w_ref, b_ref, o_ref):
    y = jnp.dot(h_ref[...], w_ref[...], preferred_element_type=jnp.float32)
    y = y + b_ref[...]
    m = jnp.max(y, axis=1, keepdims=True)
    s = jnp.sum(jnp.exp(y - m), axis=1, keepdims=True)
    o_ref[...] = (y - m) - jnp.log(s)


def _in_layer(x, W_in, b_in):
    grid = N_NODES // _ROW_BLK
    return pl.pallas_call(
        _in_layer_body,
        grid=(grid,),
        in_specs=[
            pl.BlockSpec((_ROW_BLK, NFEAT), lambda i: (i, 0)),
            pl.BlockSpec((NFEAT, NHID), lambda i: (0, 0)),
            pl.BlockSpec((1, NHID), lambda i: (0, 0)),
        ],
        out_specs=pl.BlockSpec((_ROW_BLK, NHID), lambda i: (i, 0)),
        out_shape=jax.ShapeDtypeStruct((N_NODES, NHID), jnp.float32),
    )(x, W_in, b_in.reshape(1, NHID))


def _combine(p, h0):
    grid = N_NODES // _ROW_BLK
    return pl.pallas_call(
        _combine_body,
        grid=(grid,),
        in_specs=[
            pl.BlockSpec((2, _ROW_BLK, NHID), lambda i: (0, i, 0)),
            pl.BlockSpec((_ROW_BLK, NHID), lambda i: (i, 0)),
        ],
        out_specs=pl.BlockSpec((_ROW_BLK, NHID), lambda i: (i, 0)),
        out_shape=jax.ShapeDtypeStruct((N_NODES, NHID), jnp.float32),
    )(p, h0)


def _out_layer(h, W_out, b_out):
    grid = N_NODES // _ROW_BLK
    return pl.pallas_call(
        _out_layer_body,
        grid=(grid,),
        in_specs=[
            pl.BlockSpec((_ROW_BLK, NHID), lambda i: (i, 0)),
            pl.BlockSpec((NHID, NCLASS), lambda i: (0, 0)),
            pl.BlockSpec((1, NCLASS), lambda i: (0, 0)),
        ],
        out_specs=pl.BlockSpec((_ROW_BLK, NCLASS), lambda i: (i, 0)),
        out_shape=jax.ShapeDtypeStruct((N_NODES, NCLASS), jnp.float32),
    )(h, W_out, b_out.reshape(1, NCLASS))


def _spmm_body(h_hbm, src_hbm, dst_hbm, w_hbm, out_hbm,
               src_v, dst_v, w_v, stg0, stg1, acc_sh,
               gsem0a, gsem0b, gsem1a, gsem1b, csem0, csem1):
    gsem0 = (gsem0a, gsem0b)
    gsem1 = (gsem1a, gsem1b)
    cid = lax.axis_index("c")
    sid = lax.axis_index("s")
    wid = sid * 2 + cid

    @pl.loop(0, CHUNK)
    def _zero_row(e):
        for s in range(NSLC):
            stg0[e, pl.ds(s * 16, 16)] = jnp.zeros((16,), jnp.float32)

    for k in range(ROWS_PER_TILE // CHUNK):
        pltpu.sync_copy(
            stg0, acc_sh.at[pl.ds(sid * ROWS_PER_TILE + k * CHUNK, CHUNK)])

    plsc.subcore_barrier()

    def _scale_chunk(buf, j):
        @pl.loop(0, CHUNK // 16)
        def _scale(g):
            w16 = w_v[j, pl.ds(g * 16, 16)]
            for t in range(16):
                wv = jnp.full((16,), w16[t], jnp.float32)
                e = g * 16 + t
                for s in range(NSLC):
                    sl = pl.ds(s * 16, 16)
                    buf[e, sl] = buf[e, sl] * wv

    H2 = CHUNK // 2

    class _gather:
        def __init__(self, j, buf, gsem):
            self.a = pltpu.make_async_copy(
                h_hbm.at[src_v.at[j, pl.ds(0, H2)]], buf.at[pl.ds(0, H2)],
                gsem[0])
            self.b = pltpu.make_async_copy(
                h_hbm.at[src_v.at[j, pl.ds(H2, H2)]], buf.at[pl.ds(H2, H2)],
                gsem[1])

        def start(self):
            self.a.start()
            self.b.start()

        def wait(self):
            self.a.wait()
            self.b.wait()

    def _scatter(j, buf, csem):
        return pltpu.make_async_copy(buf, acc_sh.at[dst_v.at[j]], csem)

    @pl.loop(0, NSUPER)
    def _super(b):
        sb = pl.ds(pl.multiple_of(b * SUPER, 8), SUPER)
        pltpu.sync_copy(src_hbm.at[wid, sb], src_v)
        pltpu.sync_copy(dst_hbm.at[wid, sb], dst_v)
        pltpu.sync_copy(w_hbm.at[wid, sb], w_v)

        _gather(0, stg0, gsem0).start()
        _gather(0, stg0, gsem0).wait()
        _gather(1, stg1, gsem1).start()
        _scale_chunk(stg0, 0)
        _scatter(0, stg0, csem0).start(add=True)

        @pl.loop(0, (SUPER - 2) // 2)
        def _pair(jj):
            j1 = jj * 2 + 1
            _scatter(j1 - 1, stg0, csem0).wait()
            _gather(j1 + 1, stg0, gsem0).start()
            _gather(j1, stg1, gsem1).wait()
            _scale_chunk(stg1, j1)
            _scatter(j1, stg1, csem1).start(add=True)

            j2 = j1 + 1
            _scatter(j2 - 1, stg1, csem1).wait()
            _gather(j2 + 1, stg1, gsem1).start()
            _gather(j2, stg0, gsem0).wait()
            _scale_chunk(stg0, j2)
            _scatter(j2, stg0, csem0).start(add=True)

        _scatter(SUPER - 2, stg0, csem0).wait()
        _gather(SUPER - 1, stg1, gsem1).wait()
        _scale_chunk(stg1, SUPER - 1)
        _scatter(SUPER - 1, stg1, csem1).start(add=True)
        _scatter(SUPER - 1, stg1, csem1).wait()

    plsc.subcore_barrier()

    pltpu.sync_copy(
        acc_sh.at[pl.ds(sid * ROWS_PER_TILE, ROWS_PER_TILE)],
        out_hbm.at[cid, pl.ds(sid * ROWS_PER_TILE, ROWS_PER_TILE)])


_spmm_sc = functools.partial(
    pl.kernel,
    out_type=jax.ShapeDtypeStruct((2, PAD_N, NHID), jnp.float32),
    mesh=plsc.VectorSubcoreMesh(core_axis_name="c", subcore_axis_name="s"),
    scratch_types=[
        pltpu.VMEM((SUPER, CHUNK), jnp.int32),
        pltpu.VMEM((SUPER, CHUNK), jnp.int32),
        pltpu.VMEM((SUPER, CHUNK), jnp.float32),
        pltpu.VMEM((CHUNK, NHID), jnp.float32),
        pltpu.VMEM((CHUNK, NHID), jnp.float32),
        pltpu.VMEM_SHARED((PAD_N, NHID), jnp.float32),
        pltpu.SemaphoreType.DMA,
        pltpu.SemaphoreType.DMA,
        pltpu.SemaphoreType.DMA,
        pltpu.SemaphoreType.DMA,
        pltpu.SemaphoreType.DMA,
        pltpu.SemaphoreType.DMA,
    ],
)(_spmm_body)


def kernel(x, edge_index, edge_weight, W_in, b_in, W_out, b_out):
    npad = E_PAD - N_EDGES
    iota = jnp.arange(npad, dtype=jnp.int32)
    src = jnp.concatenate(
        [edge_index[0], iota % N_NODES]).reshape(
            NW, NCHUNK, CHUNK)
    dst = jnp.concatenate(
        [edge_index[1], N_NODES + iota % (PAD_N - N_NODES)]).reshape(
            NW, NCHUNK, CHUNK)
    w = jnp.concatenate(
        [edge_weight, jnp.zeros((npad,), jnp.float32)]).reshape(
            NW, NCHUNK, CHUNK)

    h0 = _in_layer(x, W_in, b_in)
    h = h0
    for _ in range(NLAYERS):
        p = _spmm_sc(h, src, dst, w)
        h = _combine(p, h0)
    return _out_layer(h, W_out, b_out)
